# Initial kernel scaffold; baseline (speedup 1.0000x reference)
import jax
import jax.numpy as jnp
from jax.experimental import pallas as pl


def kernel(tokens, W_pos):
    raise NotImplementedError("write your pallas kernel here")


# speedup vs baseline: 1.5799x; 1.5799x over previous
import functools

import jax
import jax.numpy as jnp
from jax import lax
from jax.experimental import pallas as pl
from jax.experimental.pallas import tpu as pltpu
from jax.experimental.pallas import tpu_sc as plsc

_NUM_CORES = 2
_NUM_SUBCORES = 16
_NUM_WORKERS = _NUM_CORES * _NUM_SUBCORES


@functools.partial(jax.jit, static_argnums=(0, 1, 2))
def _pos_embed_sc(batch, seq_len, emb_dim, w_pos):
    rows_per_w = seq_len // _NUM_WORKERS
    chunk = min(rows_per_w, 64)
    n_chunks = rows_per_w // chunk

    mesh = plsc.VectorSubcoreMesh(
        core_axis_name="c", subcore_axis_name="s",
        num_cores=_NUM_CORES, num_subcores=_NUM_SUBCORES,
    )

    @functools.partial(
        pl.kernel,
        mesh=mesh,
        out_type=jax.ShapeDtypeStruct((batch * seq_len, emb_dim), jnp.float32),
        scratch_types=[
            pltpu.VMEM((chunk, emb_dim), jnp.float32),
            pltpu.SemaphoreType.DMA,
        ],
    )
    def k(w_hbm, out_hbm, buf, sem):
        wid = lax.axis_index("s") * _NUM_CORES + lax.axis_index("c")
        base = wid * rows_per_w
        for c in range(n_chunks):
            row0 = base + c * chunk
            pltpu.sync_copy(w_hbm.at[pl.ds(row0, chunk)], buf)
            copies = [
                pltpu.async_copy(
                    buf, out_hbm.at[pl.ds(b * seq_len + row0, chunk)], sem)
                for b in range(batch)
            ]
            for cp in copies:
                cp.wait()

    return k(w_pos)


def kernel(tokens, W_pos):
    batch, seq_len = tokens.shape
    emb_dim = W_pos.shape[1]
    out = _pos_embed_sc(batch, seq_len, emb_dim, W_pos)
    return out.reshape(batch, seq_len, emb_dim)
